# baseline (speedup 1.0000x reference)
import functools

import jax
import jax.numpy as jnp
from jax import lax
from jax.experimental import pallas as pl
from jax.experimental.pallas import tpu as pltpu
from jax.experimental.pallas import tpu_sc as plsc

ALPHA = 0.1
K = 10
L = 16
NC, NS = 2, 16
NW = NC * NS


def _fast_rsqrt(d):
    i = lax.bitcast_convert_type(d, jnp.int32)
    i = jnp.int32(0x5F3759DF) - lax.shift_right_arithmetic(i, 1)
    y = lax.bitcast_convert_type(i, jnp.float32)
    for _ in range(3):
        y = y * (1.5 - 0.5 * d * y * y)
    return y


def _make_deg(n, e, n_chunks):
    ch = e // n_chunks
    assert ch * n_chunks == e and ch % L == 0 and ch % 8 == 0
    mesh = plsc.VectorSubcoreMesh(
        core_axis_name="c", subcore_axis_name="s", num_cores=NC, num_subcores=NS
    )

    @functools.partial(
        pl.kernel,
        mesh=mesh,
        compiler_params=pltpu.CompilerParams(needs_layout_passes=False),
        out_type=jax.ShapeDtypeStruct((n,), jnp.float32),
        scratch_types=[
            pltpu.VMEM((n,), jnp.float32),
            pltpu.VMEM((ch,), jnp.int32),
            pltpu.VMEM((ch,), jnp.int32),
            pltpu.SemaphoreType.DMA,
            pltpu.SemaphoreType.DMA,
        ],
    )
    def deg_kernel(edge_hbm, dinv_hbm, deg_v, eb0, eb1, sem0, sem1):
        wid = lax.axis_index("s") * NC + lax.axis_index("c")
        eb = (eb0, eb1)
        sems = (sem0, sem1)
        ones = jnp.full((L,), 1.0, jnp.float32)

        @plsc.parallel_loop(0, n, step=L, unroll=4)
        def init_deg(g):
            deg_v[pl.ds(g, L)] = ones

        def start_chunk(c, p):
            return pltpu.async_copy(
                edge_hbm.at[pl.ds(c * ch, ch)], eb[p], sems[p])

        pend = start_chunk(0, 0)
        for c in range(n_chunks):
            p = c & 1
            cur = pend
            if c + 1 < n_chunks:
                pend = start_chunk(c + 1, p ^ 1)
            cur.wait()

            @plsc.parallel_loop(0, ch, step=L, unroll=10)
            def grp_body(g, _p=p):
                pv = eb[_p][pl.ds(g, L)]
                dv = lax.shift_right_logical(pv, jnp.int32(14))
                plsc.addupdate_scatter(deg_v, [dv], ones)

        @plsc.parallel_loop(0, n, step=L, unroll=2)
        def calc_dinv(g):
            s = pl.ds(g, L)
            deg_v[s] = _fast_rsqrt(deg_v[s])

        @pl.when(wid == 0)
        def _():
            pltpu.sync_copy(deg_v, dinv_hbm)

    return deg_kernel


def _make_prop(n, e, f, n_chunks):
    fp = f // NW
    ch = e // n_chunks
    assert fp * NW == f and ch * n_chunks == e and ch % L == 0 and ch % 8 == 0
    assert n <= (1 << 14)
    n_grp = n // L
    c_grp = ch // L
    mesh = plsc.VectorSubcoreMesh(
        core_axis_name="c", subcore_axis_name="s", num_cores=NC, num_subcores=NS
    )

    @functools.partial(
        pl.kernel,
        mesh=mesh,
        compiler_params=pltpu.CompilerParams(needs_layout_passes=False),
        out_type=jax.ShapeDtypeStruct((f, n), jnp.float32),
        scratch_types=(
            [pltpu.VMEM((n,), jnp.float32)]
            + [pltpu.VMEM((n,), jnp.float32)] * fp
            + [pltpu.VMEM((n,), jnp.float32)] * fp
            + [pltpu.VMEM((n,), jnp.float32)] * fp
            + [pltpu.VMEM((ch,), jnp.int32)] * 2
            + [
                pltpu.SemaphoreType.DMA,
                pltpu.SemaphoreType.DMA,
                pltpu.SemaphoreType.DMA,
            ]
        ),
    )
    def prop(h0t_hbm, edge_hbm, dinv_hbm, out_hbm, *rest):
        dinv_v = rest[0]
        h0_v = rest[1:1 + fp]
        zt_v = rest[1 + fp:1 + 2 * fp]
        acc_v = rest[1 + 2 * fp:1 + 3 * fp]
        eb0, eb1, sem0, sem1, hsem = rest[1 + 3 * fp:]
        eb = (eb0, eb1)
        wid = lax.axis_index("s") * NC + lax.axis_index("c")
        f0 = wid * fp
        sems = (sem0, sem1)

        def start_chunk(c, p):
            return pltpu.async_copy(
                edge_hbm.at[pl.ds(c * ch, ch)], eb[p], sems[p])

        assert n_chunks % 2 == 0

        def edge_pass(proc, prime_next):
            pend = None
            for c in range(n_chunks):
                p = c & 1
                cur = pend if c else chunk0_pend[0]
                if c + 1 < n_chunks:
                    pend = start_chunk(c + 1, p ^ 1)
                elif prime_next:
                    chunk0_pend[0] = start_chunk(0, 0)
                cur.wait()

                @plsc.parallel_loop(0, ch, step=L, unroll=10)
                def grp_body(g, _p=p):
                    pv = eb[_p][pl.ds(g, L)]
                    sv = lax.bitwise_and(pv, jnp.int32((1 << 14) - 1))
                    dv = lax.shift_right_logical(pv, jnp.int32(14))
                    proc(sv, dv)

        chunk0_pend = [start_chunk(0, 0)]

        in_copies = [
            pltpu.async_copy(h0t_hbm.at[f0 + j], h0_v[j], hsem)
            for j in range(fp)
        ]
        in_copies.append(pltpu.async_copy(dinv_hbm, dinv_v, hsem))

        for cp in in_copies:
            cp.wait()

        @plsc.parallel_loop(0, n, step=L, unroll=2)
        def init_zt(g):
            s = pl.ds(g, L)
            dv = dinv_v[s]
            for j in range(fp):
                zt_v[j][s] = dv * h0_v[j][s]

        zeros = jnp.zeros((L,), jnp.float32)

        @plsc.parallel_loop(0, n, step=L, unroll=4)
        def zero_acc(g):
            s = pl.ds(g, L)
            for j in range(fp):
                acc_v[j][s] = zeros

        def scatter_edges(sv, dv):
            for j in range(fp):
                vals = plsc.load_gather(zt_v[j], [sv])
                plsc.addupdate_scatter(acc_v[j], [dv], vals)

        def one_iter(last):
            edge_pass(scatter_edges, prime_next=not last)

            @plsc.parallel_loop(0, n, step=L, unroll=2)
            def upd(g):
                s = pl.ds(g, L)
                dv = dinv_v[s]
                for j in range(fp):
                    z = ((1.0 - ALPHA) * dv * (acc_v[j][s] + zt_v[j][s])
                         + ALPHA * h0_v[j][s])
                    zt_v[j][s] = z if last else dv * z
                    if not last:
                        acc_v[j][s] = zeros

        def k_body(k, c):
            one_iter(False)
            return c

        lax.fori_loop(0, K - 1, k_body, 0)
        one_iter(True)

        for j in range(fp):
            pltpu.sync_copy(zt_v[j], out_hbm.at[f0 + j])

    return prop


def _make_mm_t(m, kdim, ndim):
    def body(x_ref, w_ref, b_ref, o_ref):
        o_ref[...] = (
            lax.dot_general(
                w_ref[...], x_ref[...],
                (((0,), (1,)), ((), ())),
                preferred_element_type=jnp.float32,
            )
            + b_ref[...]
        )

    return pl.pallas_call(
        body,
        out_shape=jax.ShapeDtypeStruct((ndim, m), jnp.float32),
    )


def _make_mm_tt(m, kdim, ndim):
    def body(z_ref, w_ref, b_ref, o_ref):
        o_ref[...] = (
            lax.dot_general(
                w_ref[...], jnp.maximum(z_ref[...], 0.0),
                (((0,), (0,)), ((), ())),
                preferred_element_type=jnp.float32,
            )
            + b_ref[...]
        )

    return pl.pallas_call(
        body,
        out_shape=jax.ShapeDtypeStruct((ndim, m), jnp.float32),
    )


def kernel(x, edge_index, W1, b1, W2, b2):
    n, d_in = x.shape
    e = edge_index.shape[1]
    hid = W1.shape[1]
    d_out = W2.shape[1]

    packed = jnp.bitwise_or(
        jnp.left_shift(edge_index[1], jnp.int32(14)), edge_index[0]
    )

    mm1 = _make_mm_t(n, d_in, hid)
    mm2 = _make_mm_tt(n, hid, d_out)
    deg = _make_deg(n, e, n_chunks=10)
    prop = _make_prop(n, e, hid, n_chunks=10)
    assert hid == d_out

    dinv = deg(packed)
    h0t = mm1(x, W1, b1.reshape(hid, 1))
    z1t = prop(h0t, packed, dinv)
    h2t = mm2(z1t, W2, b2.reshape(d_out, 1))
    outt = prop(h2t, packed, dinv)
    return outt.T


# speedup vs baseline: 1.0008x; 1.0008x over previous
import functools

import jax
import jax.numpy as jnp
from jax import lax
from jax.experimental import pallas as pl
from jax.experimental.pallas import tpu as pltpu
from jax.experimental.pallas import tpu_sc as plsc

ALPHA = 0.1
K = 10
L = 16
NC, NS = 2, 16
NW = NC * NS


def _fast_rsqrt(d):
    i = lax.bitcast_convert_type(d, jnp.int32)
    i = jnp.int32(0x5F3759DF) - lax.shift_right_arithmetic(i, 1)
    y = lax.bitcast_convert_type(i, jnp.float32)
    for _ in range(3):
        y = y * (1.5 - 0.5 * d * y * y)
    return y


def _make_prop(n, e, f, n_chunks, compute_dinv):
    fp = f // NW
    ch = e // n_chunks
    assert fp * NW == f and ch * n_chunks == e and ch % L == 0 and ch % 8 == 0
    assert n <= (1 << 14)
    n_grp = n // L
    c_grp = ch // L
    mesh = plsc.VectorSubcoreMesh(
        core_axis_name="c", subcore_axis_name="s", num_cores=NC, num_subcores=NS
    )

    out_type = jax.ShapeDtypeStruct((f, n), jnp.float32)
    if compute_dinv:
        out_type = (out_type, jax.ShapeDtypeStruct((n,), jnp.float32))

    @functools.partial(
        pl.kernel,
        mesh=mesh,
        compiler_params=pltpu.CompilerParams(needs_layout_passes=False),
        out_type=out_type,
        scratch_types=(
            [pltpu.VMEM((n,), jnp.float32)]
            + [pltpu.VMEM((n,), jnp.float32)] * fp
            + [pltpu.VMEM((n,), jnp.float32)] * fp
            + [pltpu.VMEM((n,), jnp.float32)] * fp
            + [pltpu.VMEM((ch,), jnp.int32)] * 2
            + [
                pltpu.SemaphoreType.DMA,
                pltpu.SemaphoreType.DMA,
                pltpu.SemaphoreType.DMA,
            ]
        ),
    )
    def prop(h0t_hbm, edge_hbm, *rest):
        if compute_dinv:
            out_hbm, dinv_hbm = rest[0], rest[1]
        else:
            dinv_hbm, out_hbm = rest[0], rest[1]
        rest = rest[2:]
        dinv_v = rest[0]
        h0_v = rest[1:1 + fp]
        zt_v = rest[1 + fp:1 + 2 * fp]
        acc_v = rest[1 + 2 * fp:1 + 3 * fp]
        eb0, eb1, sem0, sem1, hsem = rest[1 + 3 * fp:]
        eb = (eb0, eb1)
        wid = lax.axis_index("s") * NC + lax.axis_index("c")
        f0 = wid * fp
        sems = (sem0, sem1)

        def start_chunk(c, p):
            return pltpu.async_copy(
                edge_hbm.at[pl.ds(c * ch, ch)], eb[p], sems[p])

        assert n_chunks % 2 == 0

        def edge_pass(proc, prime_next):
            pend = None
            for c in range(n_chunks):
                p = c & 1
                cur = pend if c else chunk0_pend[0]
                if c + 1 < n_chunks:
                    pend = start_chunk(c + 1, p ^ 1)
                elif prime_next:
                    chunk0_pend[0] = start_chunk(0, 0)
                cur.wait()

                @plsc.parallel_loop(0, ch, step=L, unroll=10)
                def grp_body(g, _p=p):
                    pv = eb[_p][pl.ds(g, L)]
                    sv = lax.bitwise_and(pv, jnp.int32((1 << 14) - 1))
                    dv = lax.shift_right_logical(pv, jnp.int32(14))
                    proc(sv, dv)

        chunk0_pend = [start_chunk(0, 0)]

        in_copies = [
            pltpu.async_copy(h0t_hbm.at[f0 + j], h0_v[j], hsem)
            for j in range(fp)
        ]
        if not compute_dinv:
            in_copies.append(pltpu.async_copy(dinv_hbm, dinv_v, hsem))

        ones = jnp.full((L,), 1.0, jnp.float32)

        if compute_dinv:
            @plsc.parallel_loop(0, n, step=L, unroll=4)
            def init_deg(g):
                acc_v[0][pl.ds(g, L)] = ones

            def hist(sv, dv):
                plsc.addupdate_scatter(acc_v[0], [dv], ones)

            edge_pass(hist, prime_next=True)

            @plsc.parallel_loop(0, n, step=L, unroll=2)
            def calc_dinv(g):
                s = pl.ds(g, L)
                dinv_v[s] = _fast_rsqrt(acc_v[0][s])

            @pl.when(wid == 0)
            def _():
                pltpu.sync_copy(dinv_v, dinv_hbm)

        for cp in in_copies:
            cp.wait()

        @plsc.parallel_loop(0, n, step=L, unroll=2)
        def init_zt(g):
            s = pl.ds(g, L)
            dv = dinv_v[s]
            for j in range(fp):
                zt_v[j][s] = dv * h0_v[j][s]

        zeros = jnp.zeros((L,), jnp.float32)

        @plsc.parallel_loop(0, n, step=L, unroll=4)
        def zero_acc(g):
            s = pl.ds(g, L)
            for j in range(fp):
                acc_v[j][s] = zeros

        def scatter_edges(sv, dv):
            for j in range(fp):
                vals = plsc.load_gather(zt_v[j], [sv])
                plsc.addupdate_scatter(acc_v[j], [dv], vals)

        def one_iter(last):
            edge_pass(scatter_edges, prime_next=not last)

            @plsc.parallel_loop(0, n, step=L, unroll=2)
            def upd(g):
                s = pl.ds(g, L)
                dv = dinv_v[s]
                for j in range(fp):
                    z = ((1.0 - ALPHA) * dv * (acc_v[j][s] + zt_v[j][s])
                         + ALPHA * h0_v[j][s])
                    zt_v[j][s] = z if last else dv * z
                    if not last:
                        acc_v[j][s] = zeros

        def k_body(k, c):
            one_iter(False)
            return c

        lax.fori_loop(0, K - 1, k_body, 0)
        one_iter(True)

        for j in range(fp):
            pltpu.sync_copy(zt_v[j], out_hbm.at[f0 + j])

    return prop


def _make_mm_t(m, kdim, ndim):
    def body(x_ref, w_ref, b_ref, o_ref):
        o_ref[...] = (
            lax.dot_general(
                w_ref[...], x_ref[...],
                (((0,), (1,)), ((), ())),
                preferred_element_type=jnp.float32,
            )
            + b_ref[...]
        )

    return pl.pallas_call(
        body,
        out_shape=jax.ShapeDtypeStruct((ndim, m), jnp.float32),
    )


def _make_mm_tt(m, kdim, ndim):
    def body(z_ref, w_ref, b_ref, o_ref):
        o_ref[...] = (
            lax.dot_general(
                w_ref[...], jnp.maximum(z_ref[...], 0.0),
                (((0,), (0,)), ((), ())),
                preferred_element_type=jnp.float32,
            )
            + b_ref[...]
        )

    return pl.pallas_call(
        body,
        out_shape=jax.ShapeDtypeStruct((ndim, m), jnp.float32),
    )


def kernel(x, edge_index, W1, b1, W2, b2):
    n, d_in = x.shape
    e = edge_index.shape[1]
    hid = W1.shape[1]
    d_out = W2.shape[1]

    packed = jnp.bitwise_or(
        jnp.left_shift(edge_index[1], jnp.int32(14)), edge_index[0]
    )

    mm1 = _make_mm_t(n, d_in, hid)
    mm2 = _make_mm_tt(n, hid, d_out)
    prop1 = _make_prop(n, e, hid, n_chunks=10, compute_dinv=True)
    prop2 = _make_prop(n, e, d_out, n_chunks=10, compute_dinv=False)

    h0t = mm1(x, W1, b1.reshape(hid, 1))
    z1t, dinv = prop1(h0t, packed)
    h2t = mm2(z1t, W2, b2.reshape(d_out, 1))
    outt = prop2(h2t, packed, dinv)
    return outt.T


# speedup vs baseline: 1.0067x; 1.0059x over previous
import functools

import jax
import jax.numpy as jnp
from jax import lax
from jax.experimental import pallas as pl
from jax.experimental.pallas import tpu as pltpu
from jax.experimental.pallas import tpu_sc as plsc

ALPHA = 0.1
K = 10
L = 16
NC, NS = 2, 16
NW = NC * NS


def _fast_rsqrt(d):
    i = lax.bitcast_convert_type(d, jnp.int32)
    i = jnp.int32(0x5F3759DF) - lax.shift_right_arithmetic(i, 1)
    y = lax.bitcast_convert_type(i, jnp.float32)
    for _ in range(3):
        y = y * (1.5 - 0.5 * d * y * y)
    return y


def _make_prop(n, e, f, n_chunks, compute_dinv):
    fp = f // NW
    ch = e // n_chunks
    assert fp * NW == f and ch * n_chunks == e and ch % L == 0 and ch % 8 == 0
    assert n <= (1 << 14) and n % L == 0
    mesh = plsc.VectorSubcoreMesh(
        core_axis_name="c", subcore_axis_name="s", num_cores=NC, num_subcores=NS
    )

    out_type = jax.ShapeDtypeStruct((f, n), jnp.float32)
    if compute_dinv:
        out_type = (out_type, jax.ShapeDtypeStruct((n,), jnp.float32))

    @functools.partial(
        pl.kernel,
        mesh=mesh,
        compiler_params=pltpu.CompilerParams(needs_layout_passes=False),
        out_type=out_type,
        scratch_types=(
            [pltpu.VMEM((n,), jnp.float32)]
            + [pltpu.VMEM((n,), jnp.float32)] * fp
            + [pltpu.VMEM((n,), jnp.float32)] * fp
            + [pltpu.VMEM((n,), jnp.float32)] * fp
            + [pltpu.VMEM((ch,), jnp.int32)] * 2
            + [
                pltpu.SemaphoreType.DMA,
                pltpu.SemaphoreType.DMA,
                pltpu.SemaphoreType.DMA,
            ]
        ),
    )
    def prop(h0t_hbm, edge_hbm, *rest):
        if compute_dinv:
            out_hbm, dinv_hbm = rest[0], rest[1]
        else:
            dinv_hbm, out_hbm = rest[0], rest[1]
        rest = rest[2:]
        dinv_v = rest[0]
        h0_v = rest[1:1 + fp]
        zt_v = rest[1 + fp:1 + 2 * fp]
        acc_v = rest[1 + 2 * fp:1 + 3 * fp]
        eb0, eb1, sem0, sem1, hsem = rest[1 + 3 * fp:]
        eb = (eb0, eb1)
        wid = lax.axis_index("s") * NC + lax.axis_index("c")
        f0 = wid * fp
        sems = (sem0, sem1)

        def start_chunk(c, p):
            return pltpu.async_copy(
                edge_hbm.at[pl.ds(c * ch, ch)], eb[p], sems[p])

        assert n_chunks % 2 == 0

        def edge_pass(proc, prime_next):
            pend = None
            for c in range(n_chunks):
                p = c & 1
                cur = pend if c else chunk0_pend[0]
                if c + 1 < n_chunks:
                    pend = start_chunk(c + 1, p ^ 1)
                elif prime_next:
                    chunk0_pend[0] = start_chunk(0, 0)
                cur.wait()

                @plsc.parallel_loop(0, ch, step=L, unroll=10)
                def grp_body(g, _p=p):
                    pv = eb[_p][pl.ds(g, L)]
                    sv = lax.bitwise_and(pv, jnp.int32((1 << 14) - 1))
                    dv = lax.shift_right_logical(pv, jnp.int32(14))
                    proc(sv, dv)

        chunk0_pend = [start_chunk(0, 0)]

        in_copies = [
            pltpu.async_copy(h0t_hbm.at[f0 + j], h0_v[j], hsem)
            for j in range(fp)
        ]
        if not compute_dinv:
            in_copies.append(pltpu.async_copy(dinv_hbm, dinv_v, hsem))

        ones = jnp.full((L,), 1.0, jnp.float32)

        if compute_dinv:
            @plsc.parallel_loop(0, n, step=L, unroll=4)
            def init_deg(g):
                acc_v[0][pl.ds(g, L)] = ones

            def hist(sv, dv):
                plsc.addupdate_scatter(acc_v[0], [dv], ones)

            edge_pass(hist, prime_next=True)

            @plsc.parallel_loop(0, n, step=L, unroll=2)
            def calc_dinv(g):
                s = pl.ds(g, L)
                dinv_v[s] = _fast_rsqrt(acc_v[0][s])

            @pl.when(wid == 0)
            def _():
                pltpu.sync_copy(dinv_v, dinv_hbm)

        for cp in in_copies:
            cp.wait()

        @plsc.parallel_loop(0, n, step=L, unroll=2)
        def init_zt(g):
            s = pl.ds(g, L)
            dv = dinv_v[s]
            for j in range(fp):
                zt_v[j][s] = dv * h0_v[j][s]

        zeros = jnp.zeros((L,), jnp.float32)

        @plsc.parallel_loop(0, n, step=L, unroll=4)
        def zero_acc(g):
            s = pl.ds(g, L)
            for j in range(fp):
                acc_v[j][s] = zeros

        def scatter_edges(sv, dv):
            for j in range(fp):
                vals = plsc.load_gather(zt_v[j], [sv])
                plsc.addupdate_scatter(acc_v[j], [dv], vals)

        def one_iter(last):
            edge_pass(scatter_edges, prime_next=not last)

            @plsc.parallel_loop(0, n, step=L, unroll=2)
            def upd(g):
                s = pl.ds(g, L)
                dv = dinv_v[s]
                for j in range(fp):
                    z = ((1.0 - ALPHA) * dv * (acc_v[j][s] + zt_v[j][s])
                         + ALPHA * h0_v[j][s])
                    zt_v[j][s] = z if last else dv * z
                    if not last:
                        acc_v[j][s] = zeros

        def k_body(k, c):
            one_iter(False)
            return c

        lax.fori_loop(0, K - 1, k_body, 0)
        one_iter(True)

        for j in range(fp):
            pltpu.sync_copy(zt_v[j], out_hbm.at[f0 + j])

    return prop


def _make_mm_t(m, kdim, ndim):
    def body(x_ref, w_ref, b_ref, o_ref):
        o_ref[...] = (
            lax.dot_general(
                w_ref[...], x_ref[...],
                (((0,), (1,)), ((), ())),
                preferred_element_type=jnp.float32,
            )
            + b_ref[...]
        )

    return pl.pallas_call(
        body,
        out_shape=jax.ShapeDtypeStruct((ndim, m), jnp.float32),
    )


def _make_mm_tt(m, kdim, ndim):
    def body(z_ref, w_ref, b_ref, o_ref):
        o_ref[...] = (
            lax.dot_general(
                w_ref[...], jnp.maximum(z_ref[...], 0.0),
                (((0,), (0,)), ((), ())),
                preferred_element_type=jnp.float32,
            )
            + b_ref[...]
        )

    return pl.pallas_call(
        body,
        out_shape=jax.ShapeDtypeStruct((ndim, m), jnp.float32),
    )


def kernel(x, edge_index, W1, b1, W2, b2):
    n, d_in = x.shape
    e = edge_index.shape[1]
    hid = W1.shape[1]
    d_out = W2.shape[1]

    packed = jnp.bitwise_or(
        jnp.left_shift(edge_index[1], jnp.int32(14)), edge_index[0]
    )

    mm1 = _make_mm_t(n, d_in, hid)
    mm2 = _make_mm_tt(n, hid, d_out)
    prop1 = _make_prop(n, e, hid, n_chunks=10, compute_dinv=True)
    prop2 = _make_prop(n, e, d_out, n_chunks=10, compute_dinv=False)

    h0t = mm1(x, W1, b1.reshape(hid, 1))
    z1t, dinv = prop1(h0t, packed)
    h2t = mm2(z1t, W2, b2.reshape(d_out, 1))
    outt = prop2(h2t, packed, dinv)
    return outt.T
